# baseline (speedup 1.0000x reference)
import jax
import jax.numpy as jnp
from jax import lax
from jax.experimental import pallas as pl
from jax.experimental.pallas import tpu as pltpu

B = 64
D = 64
V = 1000000
W = 128
TAIL = V % W
VMAIN = V - TAIL
MAXBLK = VMAIN // W - 1


def _sel(blks, idx_vec, tail):
    off = jnp.minimum(lax.shift_right_logical(idx_vec, 7), MAXBLK) * W
    lane = (idx_vec - off).reshape(B, 1, 1)
    iota = lax.broadcasted_iota(jnp.int32, (B, 1, W), 2)
    mask = jnp.where(iota == lane, 1.0, 0.0).astype(jnp.float32)
    rows = jnp.sum(blks * mask, axis=2)
    tl = idx_vec.reshape(B, 1) - (V - TAIL)
    iota2 = lax.broadcasted_iota(jnp.int32, (B, TAIL), 1)
    oh2 = jnp.where(iota2 == tl, 1.0, 0.0).astype(jnp.float32)
    rows = rows + lax.dot_general(oh2, tail, (((1,), (0,)), ((), ())),
                                  preferred_element_type=jnp.float32)
    return rows


def _body(cw_ref, xw_ref, cw_v, xw_v, ctail_v, xtail_v,
          cembT_ref, xembT_ref, out_ref, cblk_v, xblk_v, sem_c, sem_x):
    copies = []
    for i in range(B):
        co = pl.multiple_of(jnp.minimum(cw_ref[i] >> 7, MAXBLK) * W, W)
        xo = pl.multiple_of(jnp.minimum(xw_ref[i] >> 7, MAXBLK) * W, W)
        copies.append(pltpu.make_async_copy(
            cembT_ref.at[:, pl.ds(co, W)], cblk_v.at[i], sem_c))
        copies.append(pltpu.make_async_copy(
            xembT_ref.at[:, pl.ds(xo, W)], xblk_v.at[i], sem_x))
    for cp in copies:
        cp.start()
    for cp in copies:
        cp.wait()

    cen = _sel(cblk_v[...], cw_v[...], ctail_v[...])
    ctx = _sel(xblk_v[...], xw_v[...], xtail_v[...])
    t = lax.dot_general(ctx, cen, (((0,), (1,)), ((), ())),
                        preferred_element_type=jnp.float32)
    m = jnp.max(t, axis=1, keepdims=True)
    bv = jnp.log(jnp.sum(jnp.exp(t - m), axis=1, keepdims=True)) + m
    loss = jnp.sum(bv) * (1.0 / D) - jnp.sum(t) * (1.0 / (D * B))
    out_ref[0, 0] = loss


_tc_loss = pl.pallas_call(
    _body,
    out_shape=jax.ShapeDtypeStruct((1, 1), jnp.float32),
    in_specs=[
        pl.BlockSpec(memory_space=pltpu.MemorySpace.SMEM),
        pl.BlockSpec(memory_space=pltpu.MemorySpace.SMEM),
        pl.BlockSpec(memory_space=pltpu.MemorySpace.VMEM),
        pl.BlockSpec(memory_space=pltpu.MemorySpace.VMEM),
        pl.BlockSpec(memory_space=pltpu.MemorySpace.VMEM),
        pl.BlockSpec(memory_space=pltpu.MemorySpace.VMEM),
        pl.BlockSpec(memory_space=pltpu.MemorySpace.HBM),
        pl.BlockSpec(memory_space=pltpu.MemorySpace.HBM),
    ],
    out_specs=pl.BlockSpec(memory_space=pltpu.MemorySpace.SMEM),
    scratch_shapes=[
        pltpu.VMEM((B, D, W), jnp.float32),
        pltpu.VMEM((B, D, W), jnp.float32),
        pltpu.SemaphoreType.DMA,
        pltpu.SemaphoreType.DMA,
    ],
)


def kernel(center_word, context_word, center_emb, context_emb):
    cw = center_word.astype(jnp.int32)
    xw = context_word.astype(jnp.int32)
    ctail = center_emb[V - TAIL:]
    xtail = context_emb[V - TAIL:]
    out = _tc_loss(cw, xw, cw, xw, ctail, xtail,
                   center_emb.T, context_emb.T)
    return out[0, 0]


# speedup vs baseline: 1.0044x; 1.0044x over previous
import jax
import jax.numpy as jnp
from jax import lax
from jax.experimental import pallas as pl
from jax.experimental.pallas import tpu as pltpu

B = 64
D = 64
V = 1000000
W = 128
TAIL = V % W
VMAIN = V - TAIL
MAXBLK = VMAIN // W - 1


def _sel(blks, idx_vec, tail):
    off = jnp.minimum(lax.shift_right_logical(idx_vec, 7), MAXBLK) * W
    lane = (idx_vec - off).reshape(B, 1, 1)
    iota = lax.broadcasted_iota(jnp.int32, (B, 1, W), 2)
    mask = jnp.where(iota == lane, 1.0, 0.0).astype(jnp.float32)
    rows = jnp.sum(blks * mask, axis=2)
    tl = idx_vec.reshape(B, 1) - (V - TAIL)
    iota2 = lax.broadcasted_iota(jnp.int32, (B, TAIL), 1)
    oh2 = jnp.where(iota2 == tl, 1.0, 0.0).astype(jnp.float32)
    rows = rows + lax.dot_general(oh2, tail, (((1,), (0,)), ((), ())),
                                  preferred_element_type=jnp.float32)
    return rows


def _body(cw_ref, xw_ref, cw_v, xw_v, ctail_v, xtail_v,
          cembT_ref, xembT_ref, out_ref, cblk_v, xblk_v, sem_c, sem_x):
    copies = []
    for i in range(B):
        co = pl.multiple_of(jnp.minimum(cw_ref[i] >> 7, MAXBLK) * W, W)
        xo = pl.multiple_of(jnp.minimum(xw_ref[i] >> 7, MAXBLK) * W, W)
        copies.append(pltpu.make_async_copy(
            cembT_ref.at[:, pl.ds(co, W)], cblk_v.at[i], sem_c))
        copies.append(pltpu.make_async_copy(
            xembT_ref.at[:, pl.ds(xo, W)], xblk_v.at[i], sem_x))
    for cp in copies:
        cp.start()
    pltpu.make_async_copy(cblk_v, cblk_v, sem_c).wait()
    pltpu.make_async_copy(xblk_v, xblk_v, sem_x).wait()

    cen = _sel(cblk_v[...], cw_v[...], ctail_v[...])
    ctx = _sel(xblk_v[...], xw_v[...], xtail_v[...])
    t = lax.dot_general(ctx, cen, (((0,), (1,)), ((), ())),
                        preferred_element_type=jnp.float32)
    m = jnp.max(t, axis=1, keepdims=True)
    bv = jnp.log(jnp.sum(jnp.exp(t - m), axis=1, keepdims=True)) + m
    loss = jnp.sum(bv) * (1.0 / D) - jnp.sum(t) * (1.0 / (D * B))
    out_ref[0, 0] = loss


_tc_loss = pl.pallas_call(
    _body,
    out_shape=jax.ShapeDtypeStruct((1, 1), jnp.float32),
    in_specs=[
        pl.BlockSpec(memory_space=pltpu.MemorySpace.SMEM),
        pl.BlockSpec(memory_space=pltpu.MemorySpace.SMEM),
        pl.BlockSpec(memory_space=pltpu.MemorySpace.VMEM),
        pl.BlockSpec(memory_space=pltpu.MemorySpace.VMEM),
        pl.BlockSpec(memory_space=pltpu.MemorySpace.VMEM),
        pl.BlockSpec(memory_space=pltpu.MemorySpace.VMEM),
        pl.BlockSpec(memory_space=pltpu.MemorySpace.HBM),
        pl.BlockSpec(memory_space=pltpu.MemorySpace.HBM),
    ],
    out_specs=pl.BlockSpec(memory_space=pltpu.MemorySpace.SMEM),
    scratch_shapes=[
        pltpu.VMEM((B, D, W), jnp.float32),
        pltpu.VMEM((B, D, W), jnp.float32),
        pltpu.SemaphoreType.DMA,
        pltpu.SemaphoreType.DMA,
    ],
)


def kernel(center_word, context_word, center_emb, context_emb):
    cw = center_word.astype(jnp.int32)
    xw = context_word.astype(jnp.int32)
    ctail = center_emb[V - TAIL:]
    xtail = context_emb[V - TAIL:]
    out = _tc_loss(cw, xw, cw, xw, ctail, xtail,
                   center_emb.T, context_emb.T)
    return out[0, 0]
